# baseline (speedup 1.0000x reference)
import jax
import jax.numpy as jnp
from jax import lax
from jax.experimental import pallas as pl
from jax.experimental.pallas import tpu as pltpu
from jax.experimental.pallas import tpu_sc as plsc

N = 10000
E = 320000
D = 128
CNT = 8
EPS = 1e-5

NC = 2
NS = 16
NW = NC * NS
CB = 64
ROWS_PER_W = 160
EPAD = NW * ROWS_PER_W * CB
NDUMMY = 112
NPAD = N + NDUMMY

RB1 = 1000
RB2 = 1000


def _ln_body(x_ref, g_ref, b_ref, o_ref):
    x = x_ref[...]
    mu = jnp.mean(x, axis=1, keepdims=True)
    xc = x - mu
    var = jnp.mean(xc * xc, axis=1, keepdims=True)
    h = xc * lax.rsqrt(var + EPS) * g_ref[0:1, :] + b_ref[0:1, :]
    o_ref[...] = jnp.maximum(h, 0.0)


def _ln(x, gamma8, beta8):
    return pl.pallas_call(
        _ln_body,
        grid=(N // RB1,),
        in_specs=[
            pl.BlockSpec((RB1, D), lambda i: (i, 0)),
            pl.BlockSpec((8, D), lambda i: (0, 0)),
            pl.BlockSpec((8, D), lambda i: (0, 0)),
        ],
        out_specs=pl.BlockSpec((RB1, D), lambda i: (i, 0)),
        out_shape=jax.ShapeDtypeStruct((N, D), jnp.float32),
    )(x, gamma8, beta8)


def _edge_body(h, srcp, dstp, zf, zc, out, outc, src_v, dst_v,
               rows0, rows1, ones_v, acc_sh, cnt_sh,
               gsem0, gsem1, ssem0, ssem1):
    c = lax.axis_index("c")
    s = lax.axis_index("s")
    wid = s * NC + c
    rows = (rows0, rows1)
    gsem = (gsem0, gsem1)
    ssem = (ssem0, ssem1)

    def fill(i, carry):
        ones_v[i] = jnp.ones((CNT,), jnp.float32)
        return carry

    lax.fori_loop(0, CB, fill, 0)

    zsl = pl.ds(s * (NPAD // NS), NPAD // NS)
    pltpu.sync_copy(zf.at[zsl], acc_sh.at[zsl])
    pltpu.sync_copy(zc.at[zsl], cnt_sh.at[zsl])

    base = wid * ROWS_PER_W
    pltpu.sync_copy(srcp.at[pl.ds(base, ROWS_PER_W)], src_v)
    pltpu.sync_copy(dstp.at[pl.ds(base, ROWS_PER_W)], dst_v)
    plsc.subcore_barrier()

    def start_gather(j, b):
        pltpu.async_copy(h.at[src_v.at[j]], rows[b], gsem[b])

    def wait_gather(j, b):
        pltpu.make_async_copy(h.at[src_v.at[j]], rows[b], gsem[b]).wait()

    def start_scatter(j, b):
        pltpu.async_copy(rows[b], acc_sh.at[dst_v.at[j]], ssem[b], add=True)
        pltpu.async_copy(ones_v, cnt_sh.at[dst_v.at[j]], ssem[b], add=True)

    def wait_scatter(j, b):
        pltpu.make_async_copy(rows[b], acc_sh.at[dst_v.at[j]], ssem[b]).wait()
        pltpu.make_async_copy(ones_v, cnt_sh.at[dst_v.at[j]], ssem[b]).wait()

    start_gather(0, 0)

    def step(g, carry):
        for b in range(2):
            j = 2 * g + b
            wait_gather(j, b)

            @pl.when(j > 0)
            def _():
                wait_scatter(j - 1, 1 - b)

            @pl.when(j + 1 < ROWS_PER_W)
            def _():
                start_gather(j + 1, 1 - b)

            start_scatter(j, b)
        return carry

    lax.fori_loop(0, ROWS_PER_W // 2, step, 0)
    wait_scatter(ROWS_PER_W - 1, 1)
    plsc.subcore_barrier()

    pltpu.sync_copy(acc_sh.at[zsl], out.at[c, zsl])
    pltpu.sync_copy(cnt_sh.at[zsl], outc.at[c, zsl])


def _edge_pass(h, srcp, dstp, zf, zc):
    mesh = plsc.VectorSubcoreMesh(
        core_axis_name="c", subcore_axis_name="s",
        num_cores=NC, num_subcores=NS)
    return pl.kernel(
        _edge_body,
        out_type=(
            jax.ShapeDtypeStruct((NC, NPAD, D), jnp.float32),
            jax.ShapeDtypeStruct((NC, NPAD, CNT), jnp.float32),
        ),
        mesh=mesh,
        scratch_types=[
            pltpu.VMEM((ROWS_PER_W, CB), jnp.int32),
            pltpu.VMEM((ROWS_PER_W, CB), jnp.int32),
            pltpu.VMEM((CB, D), jnp.float32),
            pltpu.VMEM((CB, D), jnp.float32),
            pltpu.VMEM((CB, CNT), jnp.float32),
            pltpu.VMEM_SHARED((NPAD, D), jnp.float32),
            pltpu.VMEM_SHARED((NPAD, CNT), jnp.float32),
            pltpu.SemaphoreType.DMA,
            pltpu.SemaphoreType.DMA,
            pltpu.SemaphoreType.DMA,
            pltpu.SemaphoreType.DMA,
        ],
        compiler_params=pltpu.CompilerParams(use_tc_tiling_on_sc=False),
    )(h, srcp, dstp, zf, zc)


def _self_body(h_ref, wrt_ref, bl_ref, o_ref):
    o_ref[...] = (
        jnp.dot(h_ref[...], wrt_ref[...], preferred_element_type=jnp.float32)
        + bl_ref[0:1, :]
    )


def _self_mm(h, wrt, bl8):
    return pl.pallas_call(
        _self_body,
        grid=(N // RB2,),
        in_specs=[
            pl.BlockSpec((RB2, D), lambda i: (i, 0)),
            pl.BlockSpec((D, D), lambda i: (0, 0)),
            pl.BlockSpec((8, D), lambda i: (0, 0)),
        ],
        out_specs=pl.BlockSpec((RB2, D), lambda i: (i, 0)),
        out_shape=jax.ShapeDtypeStruct((N, D), jnp.float32),
    )(h, wrt, bl8)


def _mm_body(p_ref, c_ref, s_ref, wlt_ref, o_ref):
    agg = p_ref[0] + p_ref[1]
    mean = agg / jnp.maximum(c_ref[...], 1.0)
    out = (
        jnp.dot(mean, wlt_ref[...], preferred_element_type=jnp.float32)
        + s_ref[...]
    )
    o_ref[...] = out


def _combine_mm(parts, cntp, selfmm, wlt):
    return pl.pallas_call(
        _mm_body,
        grid=(N // RB2,),
        in_specs=[
            pl.BlockSpec((NC, RB2, D), lambda i: (0, i, 0)),
            pl.BlockSpec((RB2, D), lambda i: (i, 0)),
            pl.BlockSpec((RB2, D), lambda i: (i, 0)),
            pl.BlockSpec((D, D), lambda i: (0, 0)),
        ],
        out_specs=pl.BlockSpec((RB2, D), lambda i: (i, 0)),
        out_shape=jax.ShapeDtypeStruct((N, D), jnp.float32),
    )(parts, cntp, selfmm, wlt)


@jax.jit
def kernel(x, edge_index, gamma, beta, W_l, b_l, W_r):
    src = edge_index[0]
    dst = edge_index[1]
    pad = EPAD - E
    pad_ids = jnp.arange(pad, dtype=jnp.int32)
    srcp = jnp.concatenate([src, pad_ids % 577]).reshape(NW * ROWS_PER_W, CB)
    dstp = jnp.concatenate([dst, N + (pad_ids % NDUMMY)]).reshape(
        NW * ROWS_PER_W, CB)
    gamma8 = jnp.broadcast_to(gamma, (8, D))
    beta8 = jnp.broadcast_to(beta, (8, D))
    bl8 = jnp.broadcast_to(b_l, (8, D))
    zf = jnp.zeros((NPAD, D), jnp.float32)
    zc = jnp.zeros((NPAD, CNT), jnp.float32)

    h = _ln(x, gamma8, beta8)
    parts, cntp = _edge_pass(h, srcp, dstp, zf, zc)
    selfmm = _self_mm(h, W_r.T, bl8)
    cntb = jnp.broadcast_to(cntp[0, :, 0:1] + cntp[1, :, 0:1], (NPAD, D))
    return _combine_mm(parts, cntb, selfmm, W_l.T)


# speedup vs baseline: 1.0261x; 1.0261x over previous
import jax
import jax.numpy as jnp
from jax import lax
from jax.experimental import pallas as pl
from jax.experimental.pallas import tpu as pltpu
from jax.experimental.pallas import tpu_sc as plsc

N = 10000
E = 320000
D = 128
CNT = 16
EPS = 1e-5

NC = 2
NS = 16
NW = NC * NS
CB = 64
ROWS_PER_W = 160
EPAD = NW * ROWS_PER_W * CB
NDUMMY = 112
NPAD = N + NDUMMY

RB1 = 1000
RB2 = 1000


def _ln_body(x_ref, g_ref, b_ref, wrt_ref, bl_ref, oh_ref, os_ref):
    x = x_ref[...]
    mu = jnp.mean(x, axis=1, keepdims=True)
    xc = x - mu
    var = jnp.mean(xc * xc, axis=1, keepdims=True)
    h = xc * lax.rsqrt(var + EPS) * g_ref[0:1, :] + b_ref[0:1, :]
    h = jnp.maximum(h, 0.0)
    oh_ref[...] = h
    os_ref[...] = (
        jnp.dot(h, wrt_ref[...], preferred_element_type=jnp.float32)
        + bl_ref[0:1, :]
    )


def _ln(x, gamma8, beta8, wrt, bl8):
    return pl.pallas_call(
        _ln_body,
        grid=(N // RB1,),
        in_specs=[
            pl.BlockSpec((RB1, D), lambda i: (i, 0)),
            pl.BlockSpec((8, D), lambda i: (0, 0)),
            pl.BlockSpec((8, D), lambda i: (0, 0)),
            pl.BlockSpec((D, D), lambda i: (0, 0)),
            pl.BlockSpec((8, D), lambda i: (0, 0)),
        ],
        out_specs=[
            pl.BlockSpec((RB1, D), lambda i: (i, 0)),
            pl.BlockSpec((RB1, D), lambda i: (i, 0)),
        ],
        out_shape=[
            jax.ShapeDtypeStruct((N, D), jnp.float32),
            jax.ShapeDtypeStruct((N, D), jnp.float32),
        ],
    )(x, gamma8, beta8, wrt, bl8)


def _edge_body(h, srcp, dstp, zf, zc, out, outc, src_v, dst_v,
               rows0, rows1, ones_v, acc_sh, cnt_sh,
               gsem0, gsem1, ssem0, ssem1):
    c = lax.axis_index("c")
    s = lax.axis_index("s")
    wid = s * NC + c
    rows = (rows0, rows1)
    gsem = (gsem0, gsem1)
    ssem = (ssem0, ssem1)

    def fill(i, carry):
        ones_v[i] = jnp.ones((CNT,), jnp.float32)
        return carry

    lax.fori_loop(0, CB, fill, 0)

    zsl = pl.ds(s * (NPAD // NS), NPAD // NS)
    pltpu.sync_copy(zf.at[zsl], acc_sh.at[zsl])
    pltpu.sync_copy(zc.at[zsl], cnt_sh.at[zsl])

    base = wid * ROWS_PER_W
    pltpu.sync_copy(srcp.at[pl.ds(base, ROWS_PER_W)], src_v)
    pltpu.sync_copy(dstp.at[pl.ds(base, ROWS_PER_W)], dst_v)
    plsc.subcore_barrier()

    def start_gather(j, b):
        pltpu.async_copy(h.at[src_v.at[j]], rows[b], gsem[b])

    def wait_gather(j, b):
        pltpu.make_async_copy(h.at[src_v.at[j]], rows[b], gsem[b]).wait()

    def start_scatter(j, b):
        pltpu.async_copy(rows[b], acc_sh.at[dst_v.at[j]], ssem[b], add=True)
        pltpu.async_copy(ones_v, cnt_sh.at[dst_v.at[j]], ssem[b], add=True)

    def wait_scatter(j, b):
        pltpu.make_async_copy(rows[b], acc_sh.at[dst_v.at[j]], ssem[b]).wait()
        pltpu.make_async_copy(ones_v, cnt_sh.at[dst_v.at[j]], ssem[b]).wait()

    start_gather(0, 0)

    def step(g, carry):
        for b in range(2):
            j = 2 * g + b
            wait_gather(j, b)

            @pl.when(j > 0)
            def _():
                wait_scatter(j - 1, 1 - b)

            @pl.when(j + 1 < ROWS_PER_W)
            def _():
                start_gather(j + 1, 1 - b)

            start_scatter(j, b)
        return carry

    lax.fori_loop(0, ROWS_PER_W // 2, step, 0)
    wait_scatter(ROWS_PER_W - 1, 1)
    plsc.subcore_barrier()

    pltpu.sync_copy(acc_sh.at[zsl], out.at[c, zsl])
    pltpu.sync_copy(cnt_sh.at[zsl], outc.at[c, zsl])


def _edge_pass(h, srcp, dstp, zf, zc):
    mesh = plsc.VectorSubcoreMesh(
        core_axis_name="c", subcore_axis_name="s",
        num_cores=NC, num_subcores=NS)
    return pl.kernel(
        _edge_body,
        out_type=(
            jax.ShapeDtypeStruct((NC, NPAD, D), jnp.float32),
            jax.ShapeDtypeStruct((NC, NPAD, CNT), jnp.float32),
        ),
        mesh=mesh,
        scratch_types=[
            pltpu.VMEM((ROWS_PER_W, CB), jnp.int32),
            pltpu.VMEM((ROWS_PER_W, CB), jnp.int32),
            pltpu.VMEM((CB, D), jnp.float32),
            pltpu.VMEM((CB, D), jnp.float32),
            pltpu.VMEM((CB, CNT), jnp.float32),
            pltpu.VMEM_SHARED((NPAD, D), jnp.float32),
            pltpu.VMEM_SHARED((NPAD, CNT), jnp.float32),
            pltpu.SemaphoreType.DMA,
            pltpu.SemaphoreType.DMA,
            pltpu.SemaphoreType.DMA,
            pltpu.SemaphoreType.DMA,
        ],
        compiler_params=pltpu.CompilerParams(use_tc_tiling_on_sc=False),
    )(h, srcp, dstp, zf, zc)


def _mm_body(p_ref, c_ref, s_ref, wlt_ref, o_ref):
    agg = p_ref[0] + p_ref[1]
    cnt = c_ref[0][:, 0:1] + c_ref[1][:, 0:1]
    mean = agg / jnp.maximum(cnt, 1.0)
    out = (
        jnp.dot(mean, wlt_ref[...], preferred_element_type=jnp.float32)
        + s_ref[...]
    )
    o_ref[...] = out


def _combine_mm(parts, cntp, selfmm, wlt):
    return pl.pallas_call(
        _mm_body,
        grid=(N // RB2,),
        in_specs=[
            pl.BlockSpec((NC, RB2, D), lambda i: (0, i, 0)),
            pl.BlockSpec((NC, RB2, CNT), lambda i: (0, i, 0)),
            pl.BlockSpec((RB2, D), lambda i: (i, 0)),
            pl.BlockSpec((D, D), lambda i: (0, 0)),
        ],
        out_specs=pl.BlockSpec((RB2, D), lambda i: (i, 0)),
        out_shape=jax.ShapeDtypeStruct((N, D), jnp.float32),
    )(parts, cntp, selfmm, wlt)


@jax.jit
def kernel(x, edge_index, gamma, beta, W_l, b_l, W_r):
    src = edge_index[0]
    dst = edge_index[1]
    pad = EPAD - E
    pad_ids = jnp.arange(pad, dtype=jnp.int32)
    srcp = jnp.concatenate([src, pad_ids % 577]).reshape(NW * ROWS_PER_W, CB)
    dstp = jnp.concatenate([dst, N + (pad_ids % NDUMMY)]).reshape(
        NW * ROWS_PER_W, CB)
    gamma8 = jnp.broadcast_to(gamma, (8, D))
    beta8 = jnp.broadcast_to(beta, (8, D))
    bl8 = jnp.broadcast_to(b_l, (8, D))
    zf = jnp.zeros((NPAD, D), jnp.float32)
    zc = jnp.zeros((NPAD, CNT), jnp.float32)

    h, selfmm = _ln(x, gamma8, beta8, W_r.T, bl8)
    parts, cntp = _edge_pass(h, srcp, dstp, zf, zc)
    return _combine_mm(parts, cntp, selfmm, W_l.T)


# speedup vs baseline: 1.2785x; 1.2460x over previous
import jax
import jax.numpy as jnp
from jax import lax
from jax.experimental import pallas as pl
from jax.experimental.pallas import tpu as pltpu
from jax.experimental.pallas import tpu_sc as plsc

N = 10000
E = 320000
D = 128
CNT = 16
EPS = 1e-5

NC = 2
NS = 16
NW = NC * NS
CB = 64
ROWS_PER_W = 160
EPAD = NW * ROWS_PER_W * CB
NDUMMY = 112
NPAD = N + NDUMMY

RB1 = 1000
RB2 = 1000


def _ln_body(x_ref, g_ref, b_ref, o_ref):
    x = x_ref[...]
    mu = jnp.mean(x, axis=1, keepdims=True)
    xc = x - mu
    var = jnp.mean(xc * xc, axis=1, keepdims=True)
    h = xc * lax.rsqrt(var + EPS) * g_ref[0:1, :] + b_ref[0:1, :]
    o_ref[...] = jnp.maximum(h, 0.0)


def _ln(x, gamma8, beta8):
    return pl.pallas_call(
        _ln_body,
        grid=(N // RB1,),
        in_specs=[
            pl.BlockSpec((RB1, D), lambda i: (i, 0)),
            pl.BlockSpec((8, D), lambda i: (0, 0)),
            pl.BlockSpec((8, D), lambda i: (0, 0)),
        ],
        out_specs=pl.BlockSpec((RB1, D), lambda i: (i, 0)),
        out_shape=jax.ShapeDtypeStruct((N, D), jnp.float32),
    )(x, gamma8, beta8)


def _edge_body(h, srcp, dstp, zf, zc, out, outc, src_v, dst_v,
               rows0, rows1, ones_v, acc_sh, cnt_sh,
               gsem0, gsem1, ssem0, ssem1):
    c = lax.axis_index("c")
    s = lax.axis_index("s")
    wid = s * NC + c
    rows = (rows0, rows1)
    gsem = (gsem0, gsem1)
    ssem = (ssem0, ssem1)

    def fill(i, carry):
        ones_v[i] = jnp.ones((CNT,), jnp.float32)
        return carry

    lax.fori_loop(0, CB, fill, 0)

    zsl = pl.ds(s * (NPAD // NS), NPAD // NS)
    pltpu.sync_copy(zf.at[zsl], acc_sh.at[zsl])
    pltpu.sync_copy(zc.at[zsl], cnt_sh.at[zsl])

    base = wid * ROWS_PER_W
    pltpu.sync_copy(srcp.at[pl.ds(base, ROWS_PER_W)], src_v)
    pltpu.sync_copy(dstp.at[pl.ds(base, ROWS_PER_W)], dst_v)
    plsc.subcore_barrier()

    def start_gather(j, b):
        pltpu.async_copy(h.at[src_v.at[j]], rows[b], gsem[b])

    def wait_gather(j, b):
        pltpu.make_async_copy(h.at[src_v.at[j]], rows[b], gsem[b]).wait()

    def start_scatter(j, b):
        pltpu.async_copy(rows[b], acc_sh.at[dst_v.at[j]], ssem[b], add=True)
        pltpu.async_copy(ones_v, cnt_sh.at[dst_v.at[j]], ssem[b], add=True)

    def wait_scatter(j, b):
        pltpu.make_async_copy(rows[b], acc_sh.at[dst_v.at[j]], ssem[b]).wait()
        pltpu.make_async_copy(ones_v, cnt_sh.at[dst_v.at[j]], ssem[b]).wait()

    start_gather(0, 0)

    def step(g, carry):
        for b in range(2):
            j = 2 * g + b

            @pl.when(j > 0)
            def _():
                wait_scatter(j - 1, 1 - b)

            @pl.when(j + 1 < ROWS_PER_W)
            def _():
                start_gather(j + 1, 1 - b)

            wait_gather(j, b)
            start_scatter(j, b)
        return carry

    lax.fori_loop(0, ROWS_PER_W // 2, step, 0)
    wait_scatter(ROWS_PER_W - 1, 1)
    plsc.subcore_barrier()

    pltpu.sync_copy(acc_sh.at[zsl], out.at[c, zsl])
    pltpu.sync_copy(cnt_sh.at[zsl], outc.at[c, zsl])


def _edge_pass(h, srcp, dstp, zf, zc):
    mesh = plsc.VectorSubcoreMesh(
        core_axis_name="c", subcore_axis_name="s",
        num_cores=NC, num_subcores=NS)
    return pl.kernel(
        _edge_body,
        out_type=(
            jax.ShapeDtypeStruct((NC, NPAD, D), jnp.float32),
            jax.ShapeDtypeStruct((NC, NPAD, CNT), jnp.float32),
        ),
        mesh=mesh,
        scratch_types=[
            pltpu.VMEM((ROWS_PER_W, CB), jnp.int32),
            pltpu.VMEM((ROWS_PER_W, CB), jnp.int32),
            pltpu.VMEM((CB, D), jnp.float32),
            pltpu.VMEM((CB, D), jnp.float32),
            pltpu.VMEM((CB, CNT), jnp.float32),
            pltpu.VMEM_SHARED((NPAD, D), jnp.float32),
            pltpu.VMEM_SHARED((NPAD, CNT), jnp.float32),
            pltpu.SemaphoreType.DMA,
            pltpu.SemaphoreType.DMA,
            pltpu.SemaphoreType.DMA,
            pltpu.SemaphoreType.DMA,
        ],
        compiler_params=pltpu.CompilerParams(use_tc_tiling_on_sc=False),
    )(h, srcp, dstp, zf, zc)


def _self_body(h_ref, wrt_ref, bl_ref, o_ref):
    o_ref[...] = (
        jnp.dot(h_ref[...], wrt_ref[...], preferred_element_type=jnp.float32)
        + bl_ref[0:1, :]
    )


def _self_mm(h, wrt, bl8):
    return pl.pallas_call(
        _self_body,
        grid=(N // RB2,),
        in_specs=[
            pl.BlockSpec((RB2, D), lambda i: (i, 0)),
            pl.BlockSpec((D, D), lambda i: (0, 0)),
            pl.BlockSpec((8, D), lambda i: (0, 0)),
        ],
        out_specs=pl.BlockSpec((RB2, D), lambda i: (i, 0)),
        out_shape=jax.ShapeDtypeStruct((N, D), jnp.float32),
    )(h, wrt, bl8)


def _mm_body(p_ref, c_ref, s_ref, wlt_ref, o_ref):
    agg = p_ref[0] + p_ref[1]
    cnt = c_ref[0][:, 0:1] + c_ref[1][:, 0:1]
    mean = agg / jnp.maximum(cnt, 1.0)
    out = (
        jnp.dot(mean, wlt_ref[...], preferred_element_type=jnp.float32)
        + s_ref[...]
    )
    o_ref[...] = out


def _combine_mm(parts, cntp, selfmm, wlt):
    return pl.pallas_call(
        _mm_body,
        grid=(N // RB2,),
        in_specs=[
            pl.BlockSpec((NC, RB2, D), lambda i: (0, i, 0)),
            pl.BlockSpec((NC, RB2, CNT), lambda i: (0, i, 0)),
            pl.BlockSpec((RB2, D), lambda i: (i, 0)),
            pl.BlockSpec((D, D), lambda i: (0, 0)),
        ],
        out_specs=pl.BlockSpec((RB2, D), lambda i: (i, 0)),
        out_shape=jax.ShapeDtypeStruct((N, D), jnp.float32),
    )(parts, cntp, selfmm, wlt)


@jax.jit
def kernel(x, edge_index, gamma, beta, W_l, b_l, W_r):
    src = edge_index[0]
    dst = edge_index[1]
    pad = EPAD - E
    pad_ids = jnp.arange(pad, dtype=jnp.int32)
    srcp = jnp.concatenate([src, pad_ids % 577]).reshape(NW * ROWS_PER_W, CB)
    dstp = jnp.concatenate([dst, N + (pad_ids % NDUMMY)]).reshape(
        NW * ROWS_PER_W, CB)
    gamma8 = jnp.broadcast_to(gamma, (8, D))
    beta8 = jnp.broadcast_to(beta, (8, D))
    bl8 = jnp.broadcast_to(b_l, (8, D))
    zf = jnp.zeros((NPAD, D), jnp.float32)
    zc = jnp.zeros((NPAD, CNT), jnp.float32)

    h = _ln(x, gamma8, beta8)
    parts, cntp = _edge_pass(h, srcp, dstp, zf, zc)
    selfmm = _self_mm(h, W_r.T, bl8)
    return _combine_mm(parts, cntp, selfmm, W_l.T)
